# baseline (speedup 1.0000x reference)
import functools

import jax
import jax.numpy as jnp
from jax import lax
from jax.experimental import pallas as pl
from jax.experimental.pallas import tpu as pltpu
from jax.experimental.pallas import tpu_sc as plsc

_VOCAB = 1000
_DIM = 1000
_PAD = 1024
_CHUNK = 32


def _table_body(emb_ref, w_ref, b_ref, out_ref):
    out_ref[...] = lax.dot_general(
        emb_ref[...], w_ref[...],
        dimension_numbers=(((1,), (1,)), ((), ())),
        preferred_element_type=jnp.float32,
        precision=lax.Precision.HIGHEST,
    ) + b_ref[...]


def _make_gather(n_tokens: int):
    info = plsc.get_sparse_core_info()
    nw = info.num_cores * info.num_subcores
    per_w = n_tokens // nw
    assert n_tokens % (8 * nw) == 0
    n_chunks = per_w // _CHUNK
    assert per_w % (2 * _CHUNK) == 0 and _CHUNK % 8 == 0
    mesh = plsc.VectorSubcoreMesh(core_axis_name="c", subcore_axis_name="s")

    @functools.partial(
        pl.kernel,
        mesh=mesh,
        out_type=jax.ShapeDtypeStruct((n_tokens, _DIM), jnp.float32),
        scratch_types=[
            pltpu.VMEM((per_w,), jnp.int32),
            pltpu.VMEM((_CHUNK, _PAD), jnp.float32),
            pltpu.VMEM((_CHUNK, _PAD), jnp.float32),
            pltpu.SemaphoreType.DMA,
            pltpu.SemaphoreType.DMA,
            pltpu.SemaphoreType.DMA,
            pltpu.SemaphoreType.DMA,
        ],
        compiler_params=pltpu.CompilerParams(use_tc_tiling_on_sc=False),
    )
    def gather_k(table_hbm, idx_hbm, out_hbm, idx_v, buf0, buf1,
                 gsem0, gsem1, wsem0, wsem1):
        wid = lax.axis_index("s") * info.num_cores + lax.axis_index("c")
        base = wid * per_w
        pltpu.sync_copy(idx_hbm.at[pl.ds(base, per_w)], idx_v)

        def start_gather(g, buf, sem):
            off = pl.multiple_of(g * _CHUNK, 8)
            pltpu.async_copy(table_hbm.at[idx_v.at[pl.ds(off, _CHUNK)]],
                             buf, sem)

        def wait_gather(buf, sem):
            pltpu.make_async_copy(table_hbm.at[pl.ds(0, _CHUNK)], buf,
                                  sem).wait()

        def start_write(g, buf, sem):
            off = pl.multiple_of(g * _CHUNK, 8)
            pltpu.async_copy(buf.at[:, pl.ds(0, _DIM)],
                             out_hbm.at[pl.ds(base + off, _CHUNK)], sem)

        def wait_write(buf, sem):
            pltpu.make_async_copy(buf.at[:, pl.ds(0, _DIM)],
                                  out_hbm.at[pl.ds(base, _CHUNK)], sem).wait()

        start_gather(0, buf0, gsem0)
        start_gather(1, buf1, gsem1)

        def body(k, _):
            wait_gather(buf0, gsem0)
            start_write(2 * k, buf0, wsem0)
            wait_gather(buf1, gsem1)
            start_write(2 * k + 1, buf1, wsem1)

            @pl.when(k + 1 < n_chunks // 2)
            def _():
                wait_write(buf0, wsem0)
                start_gather(2 * k + 2, buf0, gsem0)
                wait_write(buf1, wsem1)
                start_gather(2 * k + 3, buf1, gsem1)
            return _

        lax.fori_loop(0, n_chunks // 2, body, None)
        wait_write(buf0, wsem0)
        wait_write(buf1, wsem1)

    return gather_k


def kernel(idx, emb, W, b):
    bsz, seq = idx.shape
    n_tokens = bsz * seq

    w_pad = jnp.pad(W, ((0, _PAD - _VOCAB), (0, 0)))
    b_pad = jnp.pad(b, (0, _PAD - _VOCAB)).reshape(1, _PAD)

    table = pl.pallas_call(
        _table_body,
        out_shape=jax.ShapeDtypeStruct((_VOCAB, _PAD), jnp.float32),
    )(emb, w_pad, b_pad)

    flat_idx = idx.reshape(n_tokens)
    out = _make_gather(n_tokens)(table, flat_idx)
    return out.reshape(bsz, seq, _DIM)


# speedup vs baseline: 1.5159x; 1.5159x over previous
import functools

import jax
import jax.numpy as jnp
from jax import lax
from jax.experimental import pallas as pl
from jax.experimental.pallas import tpu as pltpu
from jax.experimental.pallas import tpu_sc as plsc

_VOCAB = 1000
_EMB = 32
_CHUNK = 128
_BLK = 2048


def _make_gather(n_tokens: int):
    info = plsc.get_sparse_core_info()
    nw = info.num_cores * info.num_subcores
    per_w = n_tokens // nw
    assert n_tokens % (8 * nw) == 0
    n_chunks = per_w // _CHUNK
    assert per_w % (2 * _CHUNK) == 0 and _CHUNK % 8 == 0
    mesh = plsc.VectorSubcoreMesh(core_axis_name="c", subcore_axis_name="s")

    @functools.partial(
        pl.kernel,
        mesh=mesh,
        out_type=jax.ShapeDtypeStruct((n_tokens, _EMB), jnp.float32),
        scratch_types=[
            pltpu.VMEM((per_w,), jnp.int32),
            pltpu.VMEM((_CHUNK, _EMB), jnp.float32),
            pltpu.VMEM((_CHUNK, _EMB), jnp.float32),
            pltpu.SemaphoreType.DMA,
            pltpu.SemaphoreType.DMA,
            pltpu.SemaphoreType.DMA,
            pltpu.SemaphoreType.DMA,
        ],
        compiler_params=pltpu.CompilerParams(use_tc_tiling_on_sc=False),
    )
    def gather_k(emb_hbm, idx_hbm, out_hbm, idx_v, buf0, buf1,
                 gsem0, gsem1, wsem0, wsem1):
        wid = lax.axis_index("s") * info.num_cores + lax.axis_index("c")
        base = wid * per_w
        pltpu.sync_copy(idx_hbm.at[pl.ds(base, per_w)], idx_v)

        def start_gather(g, buf, sem):
            off = pl.multiple_of(g * _CHUNK, 8)
            pltpu.async_copy(emb_hbm.at[idx_v.at[pl.ds(off, _CHUNK)]],
                             buf, sem)

        def wait_gather(buf, sem):
            pltpu.make_async_copy(emb_hbm.at[pl.ds(0, _CHUNK)], buf,
                                  sem).wait()

        def start_write(g, buf, sem):
            off = pl.multiple_of(g * _CHUNK, 8)
            pltpu.async_copy(buf, out_hbm.at[pl.ds(base + off, _CHUNK)], sem)

        def wait_write(buf, sem):
            pltpu.make_async_copy(buf, out_hbm.at[pl.ds(base, _CHUNK)],
                                  sem).wait()

        start_gather(0, buf0, gsem0)
        start_gather(1, buf1, gsem1)

        def body(k, _):
            wait_gather(buf0, gsem0)
            start_write(2 * k, buf0, wsem0)
            wait_gather(buf1, gsem1)
            start_write(2 * k + 1, buf1, wsem1)

            @pl.when(k + 1 < n_chunks // 2)
            def _():
                wait_write(buf0, wsem0)
                start_gather(2 * k + 2, buf0, gsem0)
                wait_write(buf1, wsem1)
                start_gather(2 * k + 3, buf1, gsem1)
            return _

        lax.fori_loop(0, n_chunks // 2, body, None)
        wait_write(buf0, wsem0)
        wait_write(buf1, wsem1)

    return gather_k


def _proj_body(x_ref, w_ref, b_ref, out_ref):
    out_ref[...] = lax.dot_general(
        x_ref[...], w_ref[...],
        dimension_numbers=(((1,), (1,)), ((), ())),
        preferred_element_type=jnp.float32,
        precision=lax.Precision.HIGHEST,
    ) + b_ref[...]


def kernel(idx, emb, W, b):
    bsz, seq = idx.shape
    n_tokens = bsz * seq

    flat_idx = idx.reshape(n_tokens)
    x = _make_gather(n_tokens)(emb, flat_idx)

    out = pl.pallas_call(
        _proj_body,
        grid=(n_tokens // _BLK,),
        in_specs=[
            pl.BlockSpec((_BLK, _EMB), lambda i: (i, 0)),
            pl.BlockSpec((_VOCAB, _EMB), lambda i: (0, 0)),
            pl.BlockSpec((1, _VOCAB), lambda i: (0, 0)),
        ],
        out_specs=pl.BlockSpec((_BLK, _VOCAB), lambda i: (i, 0)),
        out_shape=jax.ShapeDtypeStruct((n_tokens, _VOCAB), jnp.float32),
    )(x, W, b.reshape(1, _VOCAB))
    return out.reshape(bsz, seq, _VOCAB)


# speedup vs baseline: 2.5764x; 1.6996x over previous
import functools

import jax
import jax.numpy as jnp
from jax import lax
from jax.experimental import pallas as pl
from jax.experimental.pallas import tpu as pltpu
from jax.experimental.pallas import tpu_sc as plsc

_VOCAB = 1000
_EMB = 32
_CHUNK = 128
_BLK = 2048


def _make_gather(n_tokens: int):
    info = plsc.get_sparse_core_info()
    nw = info.num_cores * info.num_subcores
    per_w = n_tokens // nw
    assert n_tokens % (8 * nw) == 0
    n_chunks = per_w // _CHUNK
    assert per_w % (2 * _CHUNK) == 0 and _CHUNK % 8 == 0
    mesh = plsc.VectorSubcoreMesh(core_axis_name="c", subcore_axis_name="s")

    @functools.partial(
        pl.kernel,
        mesh=mesh,
        out_type=jax.ShapeDtypeStruct((n_tokens, _EMB), jnp.float32),
        scratch_types=[
            pltpu.VMEM((per_w,), jnp.int32),
            pltpu.VMEM((_CHUNK, _EMB), jnp.float32),
            pltpu.VMEM((_CHUNK, _EMB), jnp.float32),
            pltpu.SemaphoreType.DMA,
            pltpu.SemaphoreType.DMA,
            pltpu.SemaphoreType.DMA,
            pltpu.SemaphoreType.DMA,
        ],
        compiler_params=pltpu.CompilerParams(use_tc_tiling_on_sc=False),
    )
    def gather_k(emb_hbm, idx_hbm, out_hbm, idx_v, buf0, buf1,
                 gsem0, gsem1, wsem0, wsem1):
        wid = lax.axis_index("s") * info.num_cores + lax.axis_index("c")
        base = wid * per_w
        pltpu.sync_copy(idx_hbm.at[pl.ds(base, per_w)], idx_v)

        def start_gather(g, buf, sem):
            off = pl.multiple_of(g * _CHUNK, 8)
            pltpu.async_copy(emb_hbm.at[idx_v.at[pl.ds(off, _CHUNK)]],
                             buf, sem)

        def wait_gather(buf, sem):
            pltpu.make_async_copy(emb_hbm.at[pl.ds(0, _CHUNK)], buf,
                                  sem).wait()

        def start_write(g, buf, sem):
            off = pl.multiple_of(g * _CHUNK, 8)
            pltpu.async_copy(buf, out_hbm.at[pl.ds(base + off, _CHUNK)], sem)

        def wait_write(buf, sem):
            pltpu.make_async_copy(buf, out_hbm.at[pl.ds(base, _CHUNK)],
                                  sem).wait()

        start_gather(0, buf0, gsem0)
        start_gather(1, buf1, gsem1)

        def body(k, _):
            wait_gather(buf0, gsem0)
            start_write(2 * k, buf0, wsem0)
            wait_gather(buf1, gsem1)
            start_write(2 * k + 1, buf1, wsem1)

            @pl.when(k + 1 < n_chunks // 2)
            def _():
                wait_write(buf0, wsem0)
                start_gather(2 * k + 2, buf0, gsem0)
                wait_write(buf1, wsem1)
                start_gather(2 * k + 3, buf1, gsem1)
            return _

        lax.fori_loop(0, n_chunks // 2, body, None)
        wait_write(buf0, wsem0)
        wait_write(buf1, wsem1)

    return gather_k


def _proj_body(x_ref, w_ref, b_ref, out_ref):
    out_ref[...] = (lax.dot_general(
        w_ref[...], x_ref[0],
        dimension_numbers=(((1,), (1,)), ((), ())),
        preferred_element_type=jnp.float32,
        precision=lax.Precision.HIGHEST,
    ) + b_ref[...])[None]


def kernel(idx, emb, W, b):
    bsz, seq = idx.shape
    n_tokens = bsz * seq

    flat_idx = idx.T.reshape(n_tokens)
    x = _make_gather(n_tokens)(emb, flat_idx)

    out = pl.pallas_call(
        _proj_body,
        grid=(seq,),
        in_specs=[
            pl.BlockSpec((1, bsz, _EMB), lambda i: (i, 0, 0)),
            pl.BlockSpec((_VOCAB, _EMB), lambda i: (0, 0)),
            pl.BlockSpec((_VOCAB, 1), lambda i: (0, 0)),
        ],
        out_specs=pl.BlockSpec((1, _VOCAB, bsz), lambda i: (i, 0, 0)),
        out_shape=jax.ShapeDtypeStruct((seq, _VOCAB, bsz), jnp.float32),
    )(x.reshape(seq, bsz, _EMB), W, b.reshape(_VOCAB, 1))
    return jnp.transpose(out, (2, 0, 1))


# speedup vs baseline: 4.4916x; 1.7434x over previous
import functools

import jax
import jax.numpy as jnp
from jax import lax
from jax.experimental import pallas as pl
from jax.experimental.pallas import tpu as pltpu
from jax.experimental.pallas import tpu_sc as plsc

_VOCAB = 1000
_EMB = 32
_CHUNK = 128
_BLK = 2048


def _make_gather(n_tokens: int):
    info = plsc.get_sparse_core_info()
    nw = info.num_cores * info.num_subcores
    per_w = n_tokens // nw
    assert n_tokens % (8 * nw) == 0
    n_chunks = per_w // _CHUNK
    assert per_w % (2 * _CHUNK) == 0 and _CHUNK % 8 == 0
    mesh = plsc.VectorSubcoreMesh(core_axis_name="c", subcore_axis_name="s")

    @functools.partial(
        pl.kernel,
        mesh=mesh,
        out_type=jax.ShapeDtypeStruct((n_tokens, _EMB), jnp.float32),
        scratch_types=[
            pltpu.VMEM((per_w,), jnp.int32),
            pltpu.VMEM((_CHUNK, _EMB), jnp.float32),
            pltpu.VMEM((_CHUNK, _EMB), jnp.float32),
            pltpu.SemaphoreType.DMA,
            pltpu.SemaphoreType.DMA,
            pltpu.SemaphoreType.DMA,
            pltpu.SemaphoreType.DMA,
        ],
        compiler_params=pltpu.CompilerParams(use_tc_tiling_on_sc=False),
    )
    def gather_k(emb_hbm, idx_hbm, out_hbm, idx_v, buf0, buf1,
                 gsem0, gsem1, wsem0, wsem1):
        wid = lax.axis_index("s") * info.num_cores + lax.axis_index("c")
        base = wid * per_w
        pltpu.sync_copy(idx_hbm.at[pl.ds(base, per_w)], idx_v)

        def start_gather(g, buf, sem):
            off = pl.multiple_of(g * _CHUNK, 8)
            pltpu.async_copy(emb_hbm.at[idx_v.at[pl.ds(off, _CHUNK)]],
                             buf, sem)

        def wait_gather(buf, sem):
            pltpu.make_async_copy(emb_hbm.at[pl.ds(0, _CHUNK)], buf,
                                  sem).wait()

        def start_write(g, buf, sem):
            off = pl.multiple_of(g * _CHUNK, 8)
            pltpu.async_copy(buf, out_hbm.at[pl.ds(base + off, _CHUNK)], sem)

        def wait_write(buf, sem):
            pltpu.make_async_copy(buf, out_hbm.at[pl.ds(base, _CHUNK)],
                                  sem).wait()

        start_gather(0, buf0, gsem0)
        start_gather(1, buf1, gsem1)

        def body(k, _):
            wait_gather(buf0, gsem0)
            start_write(2 * k, buf0, wsem0)
            wait_gather(buf1, gsem1)
            start_write(2 * k + 1, buf1, wsem1)

            @pl.when(k + 1 < n_chunks // 2)
            def _():
                wait_write(buf0, wsem0)
                start_gather(2 * k + 2, buf0, gsem0)
                wait_write(buf1, wsem1)
                start_gather(2 * k + 3, buf1, gsem1)
            return _

        lax.fori_loop(0, n_chunks // 2, body, None)
        wait_write(buf0, wsem0)
        wait_write(buf1, wsem1)

    return gather_k


def _proj_body(x_ref, w_ref, b_ref, out_ref):
    out_ref[...] = (lax.dot_general(
        w_ref[...], x_ref[0],
        dimension_numbers=(((1,), (1,)), ((), ())),
        preferred_element_type=jnp.float32,
    ) + b_ref[...])[None]


def kernel(idx, emb, W, b):
    bsz, seq = idx.shape
    n_tokens = bsz * seq

    flat_idx = idx.T.reshape(n_tokens)
    x = _make_gather(n_tokens)(emb, flat_idx)

    out = pl.pallas_call(
        _proj_body,
        grid=(seq,),
        in_specs=[
            pl.BlockSpec((1, bsz, _EMB), lambda i: (i, 0, 0)),
            pl.BlockSpec((_VOCAB, _EMB), lambda i: (0, 0)),
            pl.BlockSpec((_VOCAB, 1), lambda i: (0, 0)),
        ],
        out_specs=pl.BlockSpec((1, _VOCAB, bsz), lambda i: (i, 0, 0)),
        out_shape=jax.ShapeDtypeStruct((seq, _VOCAB, bsz), jnp.float32),
    )(x.reshape(seq, bsz, _EMB), W, b.reshape(_VOCAB, 1))
    return jnp.transpose(out, (2, 0, 1))
